# baseline (speedup 1.0000x reference)
import jax
import jax.numpy as jnp
from jax import lax
from jax.experimental import pallas as pl
from jax.experimental.pallas import tpu as pltpu

_BT = 512
_NBUF = 4


def _softmax_rows(logits):
    m = jnp.max(logits, axis=-1, keepdims=True)
    e = jnp.exp(logits - m)
    return e * (1.0 / jnp.sum(e, axis=-1, keepdims=True))


def _router_body(x_ref, w_ref, b_ref, o_ref, xbuf, ybuf, in_sem, out_sem):
    nsteps = x_ref.shape[0] // _BT
    ngroups = nsteps // _NBUF
    dn = (((1,), (1,)), ((), ()))

    def in_copy(step, slot):
        return pltpu.make_async_copy(
            x_ref.at[pl.ds(step * _BT, _BT), :], xbuf.at[slot], in_sem.at[slot])

    def out_copy(step, slot):
        return pltpu.make_async_copy(
            ybuf.at[slot], o_ref.at[pl.ds(step * _BT, _BT), :], out_sem.at[slot])

    for k in range(_NBUF):
        in_copy(k, k).start()

    w = w_ref[...]
    bias = b_ref[...]

    def group(g, carry):
        for bslot in range(_NBUF):
            i = g * _NBUF + bslot
            oslot = bslot % 2
            in_copy(i, bslot).wait()
            if bslot < 2:
                @pl.when(g > 0)
                def _():
                    out_copy(i - 2, oslot).wait()
            else:
                out_copy(i - 2, oslot).wait()
            logits = lax.dot_general(
                xbuf[bslot], w, dn, preferred_element_type=jnp.float32) + bias
            ybuf[oslot] = _softmax_rows(logits)
            out_copy(i, oslot).start()

            @pl.when(i + _NBUF < nsteps)
            def _():
                in_copy(i + _NBUF, bslot).start()
        return carry

    lax.fori_loop(0, ngroups, group, 0)
    out_copy(nsteps - 2, (nsteps - 2) % 2).wait()
    out_copy(nsteps - 1, (nsteps - 1) % 2).wait()


def kernel(x, W, b):
    T, D = x.shape
    E = W.shape[0]
    return pl.pallas_call(
        _router_body,
        in_specs=[
            pl.BlockSpec(memory_space=pl.ANY),
            pl.BlockSpec((E, D), lambda: (0, 0)),
            pl.BlockSpec((1, E), lambda: (0, 0)),
        ],
        out_specs=pl.BlockSpec(memory_space=pl.ANY),
        out_shape=jax.ShapeDtypeStruct((T, E), jnp.float32),
        scratch_shapes=[
            pltpu.VMEM((_NBUF, _BT, D), jnp.float32),
            pltpu.VMEM((2, _BT, E), jnp.float32),
            pltpu.SemaphoreType.DMA((_NBUF,)),
            pltpu.SemaphoreType.DMA((2,)),
        ],
    )(x, W, b.reshape(1, E))
